# baseline (speedup 1.0000x reference)
import jax
import jax.numpy as jnp
from jax.experimental import pallas as pl
from jax.experimental.pallas import tpu as pltpu

CB_SIZE = 1024
CB_DIM = 256


def _vq_kernel(x_ref, cb_ref, q_ref, loss_ref, perp_ref, counts_ref):
    i = pl.program_id(0)
    n = pl.num_programs(0)
    xb = x_ref[...]
    cb = cb_ref[...]
    mm = jax.lax.dot_general(xb, cb, (((1,), (1,)), ((), ())),
                             preferred_element_type=jnp.float32)
    xnorm = jnp.sum(xb * xb, axis=1, keepdims=True)
    cnorm = jnp.sum(cb * cb, axis=1, keepdims=True).T
    dist = -(xnorm - 2.0 * mm + cnorm)
    tb = xb.shape[0]
    m = jnp.max(dist, axis=1, keepdims=True)
    iota = jax.lax.broadcasted_iota(jnp.int32, (tb, CB_SIZE), 1)
    idx = jnp.min(jnp.where(dist == m, iota, CB_SIZE), axis=1)
    onehot = (iota == idx[:, None]).astype(jnp.float32)
    quant = jax.lax.dot_general(onehot, cb, (((1,), (0,)), ((), ())),
                                preferred_element_type=jnp.float32)
    q_ref[...] = quant
    d = quant - xb
    loss_ref[...] = d * d
    part = jnp.sum(onehot, axis=0, keepdims=True)

    @pl.when(i == 0)
    def _init():
        counts_ref[...] = part

    @pl.when(i > 0)
    def _acc():
        counts_ref[...] = counts_ref[...] + part

    @pl.when(i == n - 1)
    def _fin():
        total = jnp.float32(tb) * n
        prob = counts_ref[...] / total
        ent = jnp.sum(prob * jnp.log(prob + 1e-10))
        perp_ref[...] = jnp.exp(-ent).reshape(1, 1)


def kernel(x, codebook):
    shape = x.shape
    flat = x.reshape(-1, shape[-1])
    ntok = flat.shape[0]
    nb = 2
    tb = ntok // nb

    quant, loss, perp = pl.pallas_call(
        _vq_kernel,
        grid=(nb,),
        in_specs=[
            pl.BlockSpec((tb, CB_DIM), lambda i: (i, 0)),
            pl.BlockSpec((CB_SIZE, CB_DIM), lambda i: (0, 0)),
        ],
        out_specs=[
            pl.BlockSpec((tb, CB_DIM), lambda i: (i, 0)),
            pl.BlockSpec((tb, CB_DIM), lambda i: (i, 0)),
            pl.BlockSpec((1, 1), lambda i: (0, 0)),
        ],
        out_shape=[
            jax.ShapeDtypeStruct((ntok, CB_DIM), jnp.float32),
            jax.ShapeDtypeStruct((ntok, CB_DIM), jnp.float32),
            jax.ShapeDtypeStruct((1, 1), jnp.float32),
        ],
        scratch_shapes=[pltpu.VMEM((1, CB_SIZE), jnp.float32)],
    )(flat, codebook)

    return (quant.reshape(shape), loss.reshape(shape), perp[0, 0])


# speedup vs baseline: 1.1000x; 1.1000x over previous
import jax
import jax.numpy as jnp
from jax.experimental import pallas as pl
from jax.experimental.pallas import tpu as pltpu

CB_SIZE = 1024
CB_DIM = 256


def _vq_kernel(x_ref, cb_ref, q_ref, loss_ref, perp_ref, counts_ref):
    i = pl.program_id(0)
    n = pl.num_programs(0)
    xb = x_ref[...]
    cb = cb_ref[...]
    mm = jax.lax.dot_general(xb, cb, (((1,), (1,)), ((), ())),
                             preferred_element_type=jnp.float32)
    xnorm = jnp.sum(xb * xb, axis=1, keepdims=True)
    cnorm = jnp.sum(cb * cb, axis=1, keepdims=True).T
    t = xnorm - 2.0 * mm + cnorm
    tb = xb.shape[0]
    m = jnp.min(t, axis=1, keepdims=True)
    iota = jax.lax.broadcasted_iota(jnp.int32, (tb, CB_SIZE), 1)
    idx = jnp.min(jnp.where(t == m, iota, CB_SIZE), axis=1)
    onehot = (iota == idx[:, None]).astype(jnp.float32)
    quant = jax.lax.dot_general(onehot, cb, (((1,), (0,)), ((), ())),
                                preferred_element_type=jnp.float32)
    q_ref[...] = quant
    d = quant - xb
    loss_ref[...] = d * d
    ones_row = jnp.ones((1, tb), jnp.float32)
    part = jax.lax.dot_general(ones_row, onehot, (((1,), (0,)), ((), ())),
                               preferred_element_type=jnp.float32)

    @pl.when(i == 0)
    def _init():
        counts_ref[...] = part

    @pl.when(i > 0)
    def _acc():
        counts_ref[...] = counts_ref[...] + part

    @pl.when(i == n - 1)
    def _fin():
        total = jnp.float32(tb) * n
        prob = counts_ref[...] / total
        ent = jnp.sum(prob * jnp.log(prob + 1e-10))
        perp_ref[...] = jnp.exp(-ent).reshape(1, 1)


def kernel(x, codebook):
    shape = x.shape
    flat = x.reshape(-1, shape[-1])
    ntok = flat.shape[0]
    nb = 4
    tb = ntok // nb

    quant, loss, perp = pl.pallas_call(
        _vq_kernel,
        grid=(nb,),
        in_specs=[
            pl.BlockSpec((tb, CB_DIM), lambda i: (i, 0)),
            pl.BlockSpec((CB_SIZE, CB_DIM), lambda i: (0, 0)),
        ],
        out_specs=[
            pl.BlockSpec((tb, CB_DIM), lambda i: (i, 0)),
            pl.BlockSpec((tb, CB_DIM), lambda i: (i, 0)),
            pl.BlockSpec((1, 1), lambda i: (0, 0)),
        ],
        out_shape=[
            jax.ShapeDtypeStruct((ntok, CB_DIM), jnp.float32),
            jax.ShapeDtypeStruct((ntok, CB_DIM), jnp.float32),
            jax.ShapeDtypeStruct((1, 1), jnp.float32),
        ],
        scratch_shapes=[pltpu.VMEM((1, CB_SIZE), jnp.float32)],
    )(flat, codebook)

    return (quant.reshape(shape), loss.reshape(shape), perp[0, 0])
